# baseline (speedup 1.0000x reference)
import functools
import math

import jax
import jax.numpy as jnp
from jax.experimental import pallas as pl
from jax.experimental.pallas import tpu as pltpu


def _conv_kernel(x_ref, w_ref, o_ref, *, h, w, eps, gain, khkw):
    cg = x_ref.shape[1]
    hw = h * w
    k = khkw * cg
    inv_sqrt_k = 1.0 / math.sqrt(k)

    wf = w_ref[...].astype(jnp.float32)
    ssq = jnp.sum(wf * wf, axis=(0, 2), keepdims=True)
    scale = (gain * inv_sqrt_k) / (eps + jnp.sqrt(ssq) * inv_sqrt_k)
    wn = (wf * scale).astype(jnp.bfloat16)

    xb = x_ref[0].astype(jnp.bfloat16).reshape(cg, hw)
    col = jax.lax.broadcasted_iota(jnp.int32, (1, hw), 1)
    col = (col & (w - 1)) if (w & (w - 1)) == 0 else (col % w)

    acc = None
    for di in (-1, 0, 1):
        for dj in (-1, 0, 1):
            of = di * w + dj
            if of == 0:
                s = xb
            elif of > 0:
                s = jnp.concatenate(
                    [xb[:, of:], jnp.zeros((cg, of), jnp.bfloat16)], axis=1)
            else:
                s = jnp.concatenate(
                    [jnp.zeros((cg, -of), jnp.bfloat16), xb[:, :of]], axis=1)
            if dj == -1:
                s = jnp.where(col != 0, s, jnp.bfloat16(0))
            elif dj == 1:
                s = jnp.where(col != w - 1, s, jnp.bfloat16(0))
            tap = (di + 1) * 3 + (dj + 1)
            part = jnp.dot(wn[tap], s, preferred_element_type=jnp.float32)
            acc = part if acc is None else acc + part
    o_ref[0] = acc.reshape(o_ref.shape[1:])


def kernel(x, weight):
    n, cin, h, w = x.shape
    oc, cg, kh, kw = weight.shape
    khkw = kh * kw
    hw = h * w

    wt = weight.transpose(2, 3, 0, 1).reshape(khkw, oc, cg)

    kern = functools.partial(_conv_kernel, h=h, w=w, eps=1e-4, gain=1.0,
                             khkw=khkw)
    flops = 2 * n * oc * hw * cg * khkw
    cost = pl.CostEstimate(
        flops=int(flops), transcendentals=0,
        bytes_accessed=int(x.size * 4 + wt.size * 4 + n * oc * hw * 4))

    out = pl.pallas_call(
        kern,
        out_shape=jax.ShapeDtypeStruct((n, oc, h, w), jnp.float32),
        grid=(n,),
        in_specs=[
            pl.BlockSpec((1, cin, h, w), lambda i: (i, 0, 0, 0)),
            pl.BlockSpec((khkw, oc, cg), lambda i: (0, 0, 0)),
        ],
        out_specs=pl.BlockSpec((1, oc, h, w), lambda i: (i, 0, 0, 0)),
        compiler_params=pltpu.CompilerParams(
            dimension_semantics=("parallel",),
            vmem_limit_bytes=48 * 1024 * 1024),
        cost_estimate=cost,
    )(x, wt)
    return out


# speedup vs baseline: 1.7422x; 1.7422x over previous
import functools
import math

import jax
import jax.numpy as jnp
from jax.experimental import pallas as pl
from jax.experimental.pallas import tpu as pltpu


def _conv_kernel(x_ref, w_ref, o_ref, s_ref, *, h, w, eps, gain):
    cg = x_ref.shape[1]
    hw = h * w
    k = w_ref.shape[1]
    inv_sqrt_k = 1.0 / math.sqrt(k)

    wf = w_ref[...].astype(jnp.float32)
    ssq = jnp.sum(wf * wf, axis=1, keepdims=True)
    scale = (gain * inv_sqrt_k) / (eps + jnp.sqrt(ssq) * inv_sqrt_k)
    wn = (wf * scale).astype(jnp.bfloat16)

    xb = x_ref[0]
    col = jax.lax.broadcasted_iota(jnp.int32, (1, hw), 1)
    col = (col & (w - 1)) if (w & (w - 1)) == 0 else (col % w)
    xm_l = jnp.where(col != w - 1, xb, jnp.bfloat16(0))
    xm_r = jnp.where(col != 0, xb, jnp.bfloat16(0))

    for di in (-1, 0, 1):
        for dj in (-1, 0, 1):
            src = xm_l if dj == -1 else (xm_r if dj == 1 else xb)
            of = di * w + dj
            if of == 0:
                s = src
            elif of > 0:
                s = jnp.concatenate(
                    [src[:, of:], jnp.zeros((cg, of), jnp.bfloat16)], axis=1)
            else:
                s = jnp.concatenate(
                    [jnp.zeros((cg, -of), jnp.bfloat16), src[:, :of]], axis=1)
            tap = (di + 1) * 3 + (dj + 1)
            s_ref[tap * cg:(tap + 1) * cg, :] = s

    o_ref[0] = jnp.dot(wn, s_ref[...], preferred_element_type=jnp.float32)


def kernel(x, weight):
    n, cin, h, w = x.shape
    oc, cg, kh, kw = weight.shape
    khkw = kh * kw
    hw = h * w

    wt = weight.transpose(0, 2, 3, 1).reshape(oc, khkw * cg)
    x16 = x.astype(jnp.bfloat16).reshape(n, cin, hw)

    kern = functools.partial(_conv_kernel, h=h, w=w, eps=1e-4, gain=1.0)
    flops = 2 * n * oc * hw * cg * khkw
    cost = pl.CostEstimate(
        flops=int(flops), transcendentals=0,
        bytes_accessed=int(x16.size * 2 + wt.size * 4 + n * oc * hw * 4))

    out = pl.pallas_call(
        kern,
        out_shape=jax.ShapeDtypeStruct((n, oc, hw), jnp.float32),
        grid=(n,),
        in_specs=[
            pl.BlockSpec((1, cin, hw), lambda i: (i, 0, 0)),
            pl.BlockSpec((oc, khkw * cg), lambda i: (0, 0)),
        ],
        out_specs=pl.BlockSpec((1, oc, hw), lambda i: (i, 0, 0)),
        scratch_shapes=[pltpu.VMEM((khkw * cg, hw), jnp.bfloat16)],
        compiler_params=pltpu.CompilerParams(
            dimension_semantics=("parallel",),
            vmem_limit_bytes=48 * 1024 * 1024),
        cost_estimate=cost,
    )(x16, wt)
    return out.reshape(n, oc, h, w)
